# baseline (speedup 1.0000x reference)
import functools

import jax
import jax.numpy as jnp
from jax import lax
from jax.experimental import pallas as pl
from jax.experimental.pallas import tpu as pltpu
from jax.experimental.pallas import tpu_sc as plsc

NN = 10000
EE = 320000
DD = 128
VP = 144
GG = 64
OUTD = 40

NC, NS, LL = 2, 16, 16
NW = NC * NS
EPW = EE // NW
CH = 80
NCHUNK = EPW // CH
RPT = 624
REM = NN - NS * RPT
INV_SQRT_D = 1.0 / float(DD) ** 0.5


def _sc_edge_body(q_hbm, k_hbm, vp_hbm, src_hbm, dst_hbm, zero_hbm, out_hbm,
                  srcv, dstv, tq, tk, tv, acc_sp, semq, semk, semv):
    c = lax.axis_index("c")
    s = lax.axis_index("s")
    wid = c * NS + s
    base = wid * EPW

    zoff = pl.multiple_of(s * RPT, 8)
    pltpu.sync_copy(zero_hbm.at[pl.ds(zoff, RPT)],
                    acc_sp.at[pl.ds(zoff, RPT)])

    @pl.when(s == 0)
    def _():
        pltpu.sync_copy(zero_hbm.at[pl.ds(NS * RPT, REM)],
                        acc_sp.at[pl.ds(NS * RPT, REM)])

    plsc.subcore_barrier()

    def chunk_body(ci, carry):
        off = pl.multiple_of(base + ci * CH, 8)
        pltpu.sync_copy(src_hbm.at[pl.ds(off, CH)], srcv)
        pltpu.sync_copy(dst_hbm.at[pl.ds(off, CH)], dstv)
        cq = pltpu.async_copy(q_hbm.at[dstv], tq, semq)
        ck = pltpu.async_copy(k_hbm.at[srcv], tk, semk)
        cv = pltpu.async_copy(vp_hbm.at[srcv], tv, semv)
        cq.wait()
        ck.wait()
        cv.wait()

        def group_body(g, gcarry):
            lane = lax.iota(jnp.int32, LL)
            rots = [(lane + r) % LL for r in (8, 4, 2, 1)]
            lvec = jnp.zeros((LL,), jnp.float32)
            for e in range(LL):
                row = g * LL + e
                acc = tq[row, pl.ds(0, LL)] * tk[row, pl.ds(0, LL)]
                for cc in range(1, DD // LL):
                    sl = pl.ds(cc * LL, LL)
                    acc = acc + tq[row, sl] * tk[row, sl]
                for rr in rots:
                    acc = acc + acc[rr]
                lvec = jnp.where(lane == e, acc, lvec)
            sv = jnp.exp(lvec * INV_SQRT_D)
            for e in range(LL):
                row = g * LL + e
                sc = sv[jnp.full((LL,), e, jnp.int32)]
                for cc in range(VP // LL):
                    sl = pl.ds(cc * LL, LL)
                    tv[row, sl] = tv[row, sl] * sc
            return gcarry

        lax.fori_loop(0, CH // LL, group_body, 0)
        pltpu.sync_copy(tv, acc_sp.at[dstv], add=True)
        return carry

    lax.fori_loop(0, NCHUNK, chunk_body, 0)
    plsc.subcore_barrier()
    pltpu.sync_copy(acc_sp.at[pl.ds(zoff, RPT)],
                    out_hbm.at[c, pl.ds(zoff, RPT)])

    @pl.when(s == 0)
    def _():
        pltpu.sync_copy(acc_sp.at[pl.ds(NS * RPT, REM)],
                        out_hbm.at[c, pl.ds(NS * RPT, REM)])


@functools.cache
def _get_sc_edge_pass():
    mesh = plsc.VectorSubcoreMesh(
        core_axis_name="c", subcore_axis_name="s",
        num_cores=NC, num_subcores=NS,
    )
    return functools.partial(
        pl.kernel,
        out_type=jax.ShapeDtypeStruct((NC, NN, VP), jnp.float32),
        mesh=mesh,
        compiler_params=pltpu.CompilerParams(use_tc_tiling_on_sc=False),
        scratch_types=[
            pltpu.VMEM((CH,), jnp.int32),
            pltpu.VMEM((CH,), jnp.int32),
            pltpu.VMEM((CH, DD), jnp.float32),
            pltpu.VMEM((CH, DD), jnp.float32),
            pltpu.VMEM((CH, VP), jnp.float32),
            pltpu.VMEM_SHARED((NN, VP), jnp.float32),
            pltpu.SemaphoreType.DMA,
            pltpu.SemaphoreType.DMA,
            pltpu.SemaphoreType.DMA,
        ],
    )(_sc_edge_body)


def _proj_body(x_ref, wq, bq, wk, bk, wv, bv, ws, bs,
               q_ref, k_ref, vp_ref, xs_ref):
    xb = x_ref[...]
    q_ref[...] = jnp.dot(xb, wq[...], preferred_element_type=jnp.float32) + bq[...]
    k_ref[...] = jnp.dot(xb, wk[...], preferred_element_type=jnp.float32) + bk[...]
    v = jnp.dot(xb, wv[...], preferred_element_type=jnp.float32) + bv[...]
    pad = (lax.broadcasted_iota(jnp.int32, (v.shape[0], VP - DD), 1) == 0
           ).astype(jnp.float32)
    vp_ref[...] = jnp.concatenate([v, pad], axis=1)
    xs_ref[...] = jnp.dot(xb, ws[...], preferred_element_type=jnp.float32) + bs[...]


def _projections(xin, wq, bq, wk, bk, wv, bv, ws, bs):
    nb = 10
    blk = NN // nb
    wspec = pl.BlockSpec((DD, DD), lambda i: (0, 0))
    bspec = pl.BlockSpec((1, DD), lambda i: (0, 0))
    rspec = pl.BlockSpec((blk, DD), lambda i: (i, 0))
    return pl.pallas_call(
        _proj_body,
        grid=(nb,),
        in_specs=[rspec, wspec, bspec, wspec, bspec, wspec, bspec, wspec, bspec],
        out_specs=[rspec, rspec, pl.BlockSpec((blk, VP), lambda i: (i, 0)), rspec],
        out_shape=[
            jax.ShapeDtypeStruct((NN, DD), jnp.float32),
            jax.ShapeDtypeStruct((NN, DD), jnp.float32),
            jax.ShapeDtypeStruct((NN, VP), jnp.float32),
            jax.ShapeDtypeStruct((NN, DD), jnp.float32),
        ],
    )(xin, wq, bq, wk, bk, wv, bv, ws, bs)


def _combine_body(agg_ref, xs_ref, wq, bq, wk, bk, wv, bv, ws, bs,
                  h_ref, q_ref, k_ref, vp_ref, xs1_ref):
    aggs = agg_ref[0] + agg_ref[1]
    num = aggs[:, :DD]
    den = aggs[:, DD:DD + 1]
    h = jnp.maximum(num / (den + 1e-16) + xs_ref[...], 0.0)
    h_ref[...] = h
    q_ref[...] = jnp.dot(h, wq[...], preferred_element_type=jnp.float32) + bq[...]
    k_ref[...] = jnp.dot(h, wk[...], preferred_element_type=jnp.float32) + bk[...]
    v = jnp.dot(h, wv[...], preferred_element_type=jnp.float32) + bv[...]
    pad = (lax.broadcasted_iota(jnp.int32, (h.shape[0], VP - DD), 1) == 0
           ).astype(jnp.float32)
    vp_ref[...] = jnp.concatenate([v, pad], axis=1)
    xs1_ref[...] = jnp.dot(h, ws[...], preferred_element_type=jnp.float32) + bs[...]


def _combine_project(agg, xs, wq, bq, wk, bk, wv, bv, ws, bs):
    nb = 10
    blk = NN // nb
    wspec = pl.BlockSpec((DD, DD), lambda i: (0, 0))
    bspec = pl.BlockSpec((1, DD), lambda i: (0, 0))
    rspec = pl.BlockSpec((blk, DD), lambda i: (i, 0))
    aspec = pl.BlockSpec((NC, blk, VP), lambda i: (0, i, 0))
    return pl.pallas_call(
        _combine_body,
        grid=(nb,),
        in_specs=[aspec, rspec, wspec, bspec, wspec, bspec, wspec, bspec,
                  wspec, bspec],
        out_specs=[rspec, rspec, rspec,
                   pl.BlockSpec((blk, VP), lambda i: (i, 0)), rspec],
        out_shape=[
            jax.ShapeDtypeStruct((NN, DD), jnp.float32),
            jax.ShapeDtypeStruct((NN, DD), jnp.float32),
            jax.ShapeDtypeStruct((NN, DD), jnp.float32),
            jax.ShapeDtypeStruct((NN, VP), jnp.float32),
            jax.ShapeDtypeStruct((NN, DD), jnp.float32),
        ],
    )(agg, xs, wq, bq, wk, bk, wv, bv, ws, bs)


def _final_body(agg_ref, xs_ref, h1_ref, batch_ref, w1, b1, w2, b2,
                out_ref, p1_ref, p2_ref):
    i = pl.program_id(0)
    nb = pl.num_programs(0)
    aggs = agg_ref[0] + agg_ref[1]
    num = aggs[:, :DD]
    den = aggs[:, DD:DD + 1]
    h2 = jnp.maximum(num / (den + 1e-16) + xs_ref[...], 0.0)
    bb = batch_ref[0, 0]
    onehot = (bb[None, :] ==
              lax.broadcasted_iota(jnp.int32, (GG, bb.shape[0]), 0)
              ).astype(jnp.float32)
    p1c = jnp.dot(onehot, h1_ref[...], preferred_element_type=jnp.float32)
    p2c = jnp.dot(onehot, h2, preferred_element_type=jnp.float32)

    @pl.when(i == 0)
    def _():
        p1_ref[...] = p1c
        p2_ref[...] = p2c

    @pl.when(i > 0)
    def _():
        p1_ref[...] = p1_ref[...] + p1c
        p2_ref[...] = p2_ref[...] + p2c

    @pl.when(i == nb - 1)
    def _():
        hh = jnp.maximum(
            jnp.dot(p2_ref[...], w1[...], preferred_element_type=jnp.float32)
            + b1[...], 0.0)
        out_ref[...] = jnp.dot(hh, w2[...],
                               preferred_element_type=jnp.float32) + b2[...]


def _final_stage(agg, xs, h1, batch3, w1, b1, w2, b2):
    nb = 10
    blk = NN // nb
    rspec = pl.BlockSpec((blk, DD), lambda i: (i, 0))
    aspec = pl.BlockSpec((NC, blk, VP), lambda i: (0, i, 0))
    return pl.pallas_call(
        _final_body,
        grid=(nb,),
        in_specs=[aspec, rspec, rspec,
                  pl.BlockSpec((1, 1, blk), lambda i: (i, 0, 0)),
                  pl.BlockSpec((DD, DD), lambda i: (0, 0)),
                  pl.BlockSpec((1, DD), lambda i: (0, 0)),
                  pl.BlockSpec((DD, OUTD), lambda i: (0, 0)),
                  pl.BlockSpec((1, OUTD), lambda i: (0, 0))],
        out_specs=[pl.BlockSpec((GG, OUTD), lambda i: (0, 0)),
                   pl.BlockSpec((GG, DD), lambda i: (0, 0)),
                   pl.BlockSpec((GG, DD), lambda i: (0, 0))],
        out_shape=[
            jax.ShapeDtypeStruct((GG, OUTD), jnp.float32),
            jax.ShapeDtypeStruct((GG, DD), jnp.float32),
            jax.ShapeDtypeStruct((GG, DD), jnp.float32),
        ],
    )(agg, xs, h1, batch3, w1, b1, w2, b2)


def kernel(x, edge_index, batch, Wq0, bq0, Wk0, bk0, Wv0, bv0, Ws0, bs0,
           Wq1, bq1, Wk1, bk1, Wv1, bv1, Ws1, bs1, W1, b1, W2, b2):
    src = edge_index[0].astype(jnp.int32)
    dst = edge_index[1].astype(jnp.int32)
    zeros_hbm = jnp.zeros((NN, VP), jnp.float32)
    batch3 = batch.astype(jnp.int32).reshape(10, 1, NN // 10)
    r = lambda b: b.reshape(1, -1)

    q0, k0, vp0, xs0 = _projections(x, Wq0, r(bq0), Wk0, r(bk0),
                                    Wv0, r(bv0), Ws0, r(bs0))
    agg0 = _get_sc_edge_pass()(q0, k0, vp0, src, dst, zeros_hbm)
    h1, q1, k1, vp1, xs1 = _combine_project(agg0, xs0, Wq1, r(bq1), Wk1,
                                            r(bk1), Wv1, r(bv1), Ws1, r(bs1))
    agg1 = _get_sc_edge_pass()(q1, k1, vp1, src, dst, zeros_hbm)
    out, p1, p2 = _final_stage(agg1, xs1, h1, batch3, W1, r(b1), W2, r(b2))
    return (out, p1, p2)


# speedup vs baseline: 1.1516x; 1.1516x over previous
import functools

import jax
import jax.numpy as jnp
from jax import lax
from jax.experimental import pallas as pl
from jax.experimental.pallas import tpu as pltpu
from jax.experimental.pallas import tpu_sc as plsc

NN = 10000
EE = 320000
DD = 128
VP = 144
GG = 64
OUTD = 40

NC, NS, LL = 2, 16, 16
NW = NC * NS
EPW = EE // NW
CH = 80
NCHUNK = EPW // CH
RPT = 624
REM = NN - NS * RPT
INV_SQRT_D = 1.0 / float(DD) ** 0.5


def _sc_edge_body(q_hbm, k_hbm, vp_hbm, src_hbm, dst_hbm, zero_hbm, out_hbm,
                  srcv, dstv, tq, tk, tv, acc_sp, semq, semk, semv, semi):
    c = lax.axis_index("c")
    s = lax.axis_index("s")
    wid = c * NS + s
    base = wid * EPW

    def issue_idx(ci, sl):
        off = pl.multiple_of(base + ci * CH, 8)
        pltpu.async_copy(src_hbm.at[pl.ds(off, CH)], srcv.at[sl], semi)
        pltpu.async_copy(dst_hbm.at[pl.ds(off, CH)], dstv.at[sl], semi)

    def drain_idx(ci, sl):
        off = pl.multiple_of(base + ci * CH, 8)
        pltpu.make_async_copy(src_hbm.at[pl.ds(off, CH)], srcv.at[sl],
                              semi).wait()
        pltpu.make_async_copy(dst_hbm.at[pl.ds(off, CH)], dstv.at[sl],
                              semi).wait()

    issue_idx(0, 0)

    zoff = pl.multiple_of(s * RPT, 8)
    pltpu.sync_copy(zero_hbm.at[pl.ds(zoff, RPT)],
                    acc_sp.at[pl.ds(zoff, RPT)])

    @pl.when(s == 0)
    def _():
        pltpu.sync_copy(zero_hbm.at[pl.ds(NS * RPT, REM)],
                        acc_sp.at[pl.ds(NS * RPT, REM)])

    plsc.subcore_barrier()

    def chunk_body(ci, carry):
        islot = lax.rem(ci, 2)
        drain_idx(ci, islot)
        cq = pltpu.async_copy(q_hbm.at[dstv.at[islot]], tq, semq)
        ck = pltpu.async_copy(k_hbm.at[srcv.at[islot]], tk, semk)
        cv = pltpu.async_copy(vp_hbm.at[srcv.at[islot]], tv, semv)

        @pl.when(ci + 1 < NCHUNK)
        def _():
            issue_idx(ci + 1, 1 - islot)

        cq.wait()
        ck.wait()
        cv.wait()

        def group_body(g, gcarry):
            lane = lax.iota(jnp.int32, LL)
            rots = [(lane + r) % LL for r in (8, 4, 2, 1)]
            lvec = jnp.zeros((LL,), jnp.float32)
            for e in range(LL):
                row = g * LL + e
                acc = tq[row, pl.ds(0, LL)] * tk[row, pl.ds(0, LL)]
                for cc in range(1, DD // LL):
                    sl = pl.ds(cc * LL, LL)
                    acc = acc + tq[row, sl] * tk[row, sl]
                for rr in rots:
                    acc = acc + acc[rr]
                lvec = jnp.where(lane == e, acc, lvec)
            sv = jnp.exp(lvec * INV_SQRT_D)
            for e in range(LL):
                row = g * LL + e
                sc = sv[jnp.full((LL,), e, jnp.int32)]
                for cc in range(VP // LL):
                    sl = pl.ds(cc * LL, LL)
                    tv[row, sl] = tv[row, sl] * sc
            return gcarry

        lax.fori_loop(0, CH // LL, group_body, 0)
        pltpu.sync_copy(tv, acc_sp.at[dstv.at[islot]], add=True)
        return carry

    lax.fori_loop(0, NCHUNK, chunk_body, 0)
    plsc.subcore_barrier()
    pltpu.sync_copy(acc_sp.at[pl.ds(zoff, RPT)],
                    out_hbm.at[c, pl.ds(zoff, RPT)])

    @pl.when(s == 0)
    def _():
        pltpu.sync_copy(acc_sp.at[pl.ds(NS * RPT, REM)],
                        out_hbm.at[c, pl.ds(NS * RPT, REM)])


@functools.cache
def _get_sc_edge_pass():
    mesh = plsc.VectorSubcoreMesh(
        core_axis_name="c", subcore_axis_name="s",
        num_cores=NC, num_subcores=NS,
    )
    return functools.partial(
        pl.kernel,
        out_type=jax.ShapeDtypeStruct((NC, NN, VP), jnp.float32),
        mesh=mesh,
        compiler_params=pltpu.CompilerParams(use_tc_tiling_on_sc=False),
        scratch_types=[
            pltpu.VMEM((2, CH), jnp.int32),
            pltpu.VMEM((2, CH), jnp.int32),
            pltpu.VMEM((CH, DD), jnp.float32),
            pltpu.VMEM((CH, DD), jnp.float32),
            pltpu.VMEM((CH, VP), jnp.float32),
            pltpu.VMEM_SHARED((NN, VP), jnp.float32),
            pltpu.SemaphoreType.DMA,
            pltpu.SemaphoreType.DMA,
            pltpu.SemaphoreType.DMA,
            pltpu.SemaphoreType.DMA,
        ],
    )(_sc_edge_body)


def _proj_body(x_ref, wq, bq, wk, bk, wv, bv, ws, bs,
               q_ref, k_ref, vp_ref, xs_ref):
    xb = x_ref[...]
    q_ref[...] = jnp.dot(xb, wq[...], preferred_element_type=jnp.float32) + bq[...]
    k_ref[...] = jnp.dot(xb, wk[...], preferred_element_type=jnp.float32) + bk[...]
    v = jnp.dot(xb, wv[...], preferred_element_type=jnp.float32) + bv[...]
    pad = (lax.broadcasted_iota(jnp.int32, (v.shape[0], VP - DD), 1) == 0
           ).astype(jnp.float32)
    vp_ref[...] = jnp.concatenate([v, pad], axis=1)
    xs_ref[...] = jnp.dot(xb, ws[...], preferred_element_type=jnp.float32) + bs[...]


def _projections(xin, wq, bq, wk, bk, wv, bv, ws, bs):
    nb = 10
    blk = NN // nb
    wspec = pl.BlockSpec((DD, DD), lambda i: (0, 0))
    bspec = pl.BlockSpec((1, DD), lambda i: (0, 0))
    rspec = pl.BlockSpec((blk, DD), lambda i: (i, 0))
    return pl.pallas_call(
        _proj_body,
        grid=(nb,),
        in_specs=[rspec, wspec, bspec, wspec, bspec, wspec, bspec, wspec, bspec],
        out_specs=[rspec, rspec, pl.BlockSpec((blk, VP), lambda i: (i, 0)), rspec],
        out_shape=[
            jax.ShapeDtypeStruct((NN, DD), jnp.float32),
            jax.ShapeDtypeStruct((NN, DD), jnp.float32),
            jax.ShapeDtypeStruct((NN, VP), jnp.float32),
            jax.ShapeDtypeStruct((NN, DD), jnp.float32),
        ],
    )(xin, wq, bq, wk, bk, wv, bv, ws, bs)


def _combine_body(agg_ref, xs_ref, wq, bq, wk, bk, wv, bv, ws, bs,
                  h_ref, q_ref, k_ref, vp_ref, xs1_ref):
    aggs = agg_ref[0] + agg_ref[1]
    num = aggs[:, :DD]
    den = aggs[:, DD:DD + 1]
    h = jnp.maximum(num / (den + 1e-16) + xs_ref[...], 0.0)
    h_ref[...] = h
    q_ref[...] = jnp.dot(h, wq[...], preferred_element_type=jnp.float32) + bq[...]
    k_ref[...] = jnp.dot(h, wk[...], preferred_element_type=jnp.float32) + bk[...]
    v = jnp.dot(h, wv[...], preferred_element_type=jnp.float32) + bv[...]
    pad = (lax.broadcasted_iota(jnp.int32, (h.shape[0], VP - DD), 1) == 0
           ).astype(jnp.float32)
    vp_ref[...] = jnp.concatenate([v, pad], axis=1)
    xs1_ref[...] = jnp.dot(h, ws[...], preferred_element_type=jnp.float32) + bs[...]


def _combine_project(agg, xs, wq, bq, wk, bk, wv, bv, ws, bs):
    nb = 10
    blk = NN // nb
    wspec = pl.BlockSpec((DD, DD), lambda i: (0, 0))
    bspec = pl.BlockSpec((1, DD), lambda i: (0, 0))
    rspec = pl.BlockSpec((blk, DD), lambda i: (i, 0))
    aspec = pl.BlockSpec((NC, blk, VP), lambda i: (0, i, 0))
    return pl.pallas_call(
        _combine_body,
        grid=(nb,),
        in_specs=[aspec, rspec, wspec, bspec, wspec, bspec, wspec, bspec,
                  wspec, bspec],
        out_specs=[rspec, rspec, rspec,
                   pl.BlockSpec((blk, VP), lambda i: (i, 0)), rspec],
        out_shape=[
            jax.ShapeDtypeStruct((NN, DD), jnp.float32),
            jax.ShapeDtypeStruct((NN, DD), jnp.float32),
            jax.ShapeDtypeStruct((NN, DD), jnp.float32),
            jax.ShapeDtypeStruct((NN, VP), jnp.float32),
            jax.ShapeDtypeStruct((NN, DD), jnp.float32),
        ],
    )(agg, xs, wq, bq, wk, bk, wv, bv, ws, bs)


def _final_body(agg_ref, xs_ref, h1_ref, batch_ref, w1, b1, w2, b2,
                out_ref, p1_ref, p2_ref):
    i = pl.program_id(0)
    nb = pl.num_programs(0)
    aggs = agg_ref[0] + agg_ref[1]
    num = aggs[:, :DD]
    den = aggs[:, DD:DD + 1]
    h2 = jnp.maximum(num / (den + 1e-16) + xs_ref[...], 0.0)
    bb = batch_ref[0, 0]
    onehot = (bb[None, :] ==
              lax.broadcasted_iota(jnp.int32, (GG, bb.shape[0]), 0)
              ).astype(jnp.float32)
    p1c = jnp.dot(onehot, h1_ref[...], preferred_element_type=jnp.float32)
    p2c = jnp.dot(onehot, h2, preferred_element_type=jnp.float32)

    @pl.when(i == 0)
    def _():
        p1_ref[...] = p1c
        p2_ref[...] = p2c

    @pl.when(i > 0)
    def _():
        p1_ref[...] = p1_ref[...] + p1c
        p2_ref[...] = p2_ref[...] + p2c

    @pl.when(i == nb - 1)
    def _():
        hh = jnp.maximum(
            jnp.dot(p2_ref[...], w1[...], preferred_element_type=jnp.float32)
            + b1[...], 0.0)
        out_ref[...] = jnp.dot(hh, w2[...],
                               preferred_element_type=jnp.float32) + b2[...]


def _final_stage(agg, xs, h1, batch3, w1, b1, w2, b2):
    nb = 10
    blk = NN // nb
    rspec = pl.BlockSpec((blk, DD), lambda i: (i, 0))
    aspec = pl.BlockSpec((NC, blk, VP), lambda i: (0, i, 0))
    return pl.pallas_call(
        _final_body,
        grid=(nb,),
        in_specs=[aspec, rspec, rspec,
                  pl.BlockSpec((1, 1, blk), lambda i: (i, 0, 0)),
                  pl.BlockSpec((DD, DD), lambda i: (0, 0)),
                  pl.BlockSpec((1, DD), lambda i: (0, 0)),
                  pl.BlockSpec((DD, OUTD), lambda i: (0, 0)),
                  pl.BlockSpec((1, OUTD), lambda i: (0, 0))],
        out_specs=[pl.BlockSpec((GG, OUTD), lambda i: (0, 0)),
                   pl.BlockSpec((GG, DD), lambda i: (0, 0)),
                   pl.BlockSpec((GG, DD), lambda i: (0, 0))],
        out_shape=[
            jax.ShapeDtypeStruct((GG, OUTD), jnp.float32),
            jax.ShapeDtypeStruct((GG, DD), jnp.float32),
            jax.ShapeDtypeStruct((GG, DD), jnp.float32),
        ],
    )(agg, xs, h1, batch3, w1, b1, w2, b2)


def kernel(x, edge_index, batch, Wq0, bq0, Wk0, bk0, Wv0, bv0, Ws0, bs0,
           Wq1, bq1, Wk1, bk1, Wv1, bv1, Ws1, bs1, W1, b1, W2, b2):
    src = edge_index[0].astype(jnp.int32)
    dst = edge_index[1].astype(jnp.int32)
    zeros_hbm = jnp.zeros((NN, VP), jnp.float32)
    batch3 = batch.astype(jnp.int32).reshape(10, 1, NN // 10)
    r = lambda b: b.reshape(1, -1)

    q0, k0, vp0, xs0 = _projections(x, Wq0, r(bq0), Wk0, r(bk0),
                                    Wv0, r(bv0), Ws0, r(bs0))
    agg0 = _get_sc_edge_pass()(q0, k0, vp0, src, dst, zeros_hbm)
    h1, q1, k1, vp1, xs1 = _combine_project(agg0, xs0, Wq1, r(bq1), Wk1,
                                            r(bk1), Wv1, r(bv1), Ws1, r(bs1))
    agg1 = _get_sc_edge_pass()(q1, k1, vp1, src, dst, zeros_hbm)
    out, p1, p2 = _final_stage(agg1, xs1, h1, batch3, W1, r(b1), W2, r(b2))
    return (out, p1, p2)


# speedup vs baseline: 1.2395x; 1.0763x over previous
import functools

import jax
import jax.numpy as jnp
from jax import lax
from jax.experimental import pallas as pl
from jax.experimental.pallas import tpu as pltpu
from jax.experimental.pallas import tpu_sc as plsc

NN = 10000
EE = 320000
DD = 128
VP = 144
GG = 64
OUTD = 40

NC, NS, LL = 2, 16, 16
NW = NC * NS
EPW = EE // NW
CH = 80
NCHUNK = EPW // CH
RPT = 624
REM = NN - NS * RPT
INV_SQRT_D = 1.0 / float(DD) ** 0.5


def _sc_edge_body(q_hbm, k_hbm, vp_hbm, src_hbm, dst_hbm, zero_hbm, out_hbm,
                  srcv, dstv, tq, tk, tv, acc_sp, semq, semk, semv, semi,
                  semsc):
    c = lax.axis_index("c")
    s = lax.axis_index("s")
    wid = c * NS + s
    base = wid * EPW

    def issue_idx(ci, sl):
        off = pl.multiple_of(base + ci * CH, 8)
        pltpu.async_copy(src_hbm.at[pl.ds(off, CH)], srcv.at[sl], semi)
        pltpu.async_copy(dst_hbm.at[pl.ds(off, CH)], dstv.at[sl], semi)

    def drain_idx(ci, sl):
        off = pl.multiple_of(base + ci * CH, 8)
        pltpu.make_async_copy(src_hbm.at[pl.ds(off, CH)], srcv.at[sl],
                              semi).wait()
        pltpu.make_async_copy(dst_hbm.at[pl.ds(off, CH)], dstv.at[sl],
                              semi).wait()

    issue_idx(0, 0)

    zoff = pl.multiple_of(s * RPT, 8)
    pltpu.sync_copy(zero_hbm.at[pl.ds(zoff, RPT)],
                    acc_sp.at[pl.ds(zoff, RPT)])

    @pl.when(s == 0)
    def _():
        pltpu.sync_copy(zero_hbm.at[pl.ds(NS * RPT, REM)],
                        acc_sp.at[pl.ds(NS * RPT, REM)])

    plsc.subcore_barrier()

    def chunk_body(ci, carry):
        islot = lax.rem(ci, 2)
        drain_idx(ci, islot)
        cq = pltpu.async_copy(q_hbm.at[dstv.at[islot]], tq, semq)
        ck = pltpu.async_copy(k_hbm.at[srcv.at[islot]], tk, semk)

        @pl.when(ci > 0)
        def _():
            pltpu.make_async_copy(tv, acc_sp.at[dstv.at[islot]], semsc).wait()

        cv = pltpu.async_copy(vp_hbm.at[srcv.at[islot]], tv, semv)

        @pl.when(ci + 1 < NCHUNK)
        def _():
            issue_idx(ci + 1, 1 - islot)

        cq.wait()
        ck.wait()
        cv.wait()

        def group_body(g, gcarry):
            lane = lax.iota(jnp.int32, LL)
            rots = [(lane + r) % LL for r in (8, 4, 2, 1)]
            lvec = jnp.zeros((LL,), jnp.float32)
            for e in range(LL):
                row = g * LL + e
                acc = tq[row, pl.ds(0, LL)] * tk[row, pl.ds(0, LL)]
                for cc in range(1, DD // LL):
                    sl = pl.ds(cc * LL, LL)
                    acc = acc + tq[row, sl] * tk[row, sl]
                for rr in rots:
                    acc = acc + acc[rr]
                lvec = jnp.where(lane == e, acc, lvec)
            sv = jnp.exp(lvec * INV_SQRT_D)
            for e in range(LL):
                row = g * LL + e
                sc = sv[jnp.full((LL,), e, jnp.int32)]
                for cc in range(VP // LL):
                    sl = pl.ds(cc * LL, LL)
                    tv[row, sl] = tv[row, sl] * sc
            return gcarry

        lax.fori_loop(0, CH // LL, group_body, 0)
        pltpu.async_copy(tv, acc_sp.at[dstv.at[islot]], semsc, add=True)
        return carry

    lax.fori_loop(0, NCHUNK, chunk_body, 0)
    pltpu.make_async_copy(tv, acc_sp.at[dstv.at[0]], semsc).wait()
    plsc.subcore_barrier()
    pltpu.sync_copy(acc_sp.at[pl.ds(zoff, RPT)],
                    out_hbm.at[c, pl.ds(zoff, RPT)])

    @pl.when(s == 0)
    def _():
        pltpu.sync_copy(acc_sp.at[pl.ds(NS * RPT, REM)],
                        out_hbm.at[c, pl.ds(NS * RPT, REM)])


@functools.cache
def _get_sc_edge_pass():
    mesh = plsc.VectorSubcoreMesh(
        core_axis_name="c", subcore_axis_name="s",
        num_cores=NC, num_subcores=NS,
    )
    return functools.partial(
        pl.kernel,
        out_type=jax.ShapeDtypeStruct((NC, NN, VP), jnp.float32),
        mesh=mesh,
        compiler_params=pltpu.CompilerParams(use_tc_tiling_on_sc=False),
        scratch_types=[
            pltpu.VMEM((2, CH), jnp.int32),
            pltpu.VMEM((2, CH), jnp.int32),
            pltpu.VMEM((CH, DD), jnp.float32),
            pltpu.VMEM((CH, DD), jnp.float32),
            pltpu.VMEM((CH, VP), jnp.float32),
            pltpu.VMEM_SHARED((NN, VP), jnp.float32),
            pltpu.SemaphoreType.DMA,
            pltpu.SemaphoreType.DMA,
            pltpu.SemaphoreType.DMA,
            pltpu.SemaphoreType.DMA,
            pltpu.SemaphoreType.DMA,
        ],
    )(_sc_edge_body)


def _proj_body(x_ref, wq, bq, wk, bk, wv, bv, ws, bs,
               q_ref, k_ref, vp_ref, xs_ref):
    xb = x_ref[...]
    q_ref[...] = jnp.dot(xb, wq[...], preferred_element_type=jnp.float32) + bq[...]
    k_ref[...] = jnp.dot(xb, wk[...], preferred_element_type=jnp.float32) + bk[...]
    v = jnp.dot(xb, wv[...], preferred_element_type=jnp.float32) + bv[...]
    pad = (lax.broadcasted_iota(jnp.int32, (v.shape[0], VP - DD), 1) == 0
           ).astype(jnp.float32)
    vp_ref[...] = jnp.concatenate([v, pad], axis=1)
    xs_ref[...] = jnp.dot(xb, ws[...], preferred_element_type=jnp.float32) + bs[...]


def _projections(xin, wq, bq, wk, bk, wv, bv, ws, bs):
    nb = 10
    blk = NN // nb
    wspec = pl.BlockSpec((DD, DD), lambda i: (0, 0))
    bspec = pl.BlockSpec((1, DD), lambda i: (0, 0))
    rspec = pl.BlockSpec((blk, DD), lambda i: (i, 0))
    return pl.pallas_call(
        _proj_body,
        grid=(nb,),
        in_specs=[rspec, wspec, bspec, wspec, bspec, wspec, bspec, wspec, bspec],
        out_specs=[rspec, rspec, pl.BlockSpec((blk, VP), lambda i: (i, 0)), rspec],
        out_shape=[
            jax.ShapeDtypeStruct((NN, DD), jnp.float32),
            jax.ShapeDtypeStruct((NN, DD), jnp.float32),
            jax.ShapeDtypeStruct((NN, VP), jnp.float32),
            jax.ShapeDtypeStruct((NN, DD), jnp.float32),
        ],
    )(xin, wq, bq, wk, bk, wv, bv, ws, bs)


def _combine_body(agg_ref, xs_ref, wq, bq, wk, bk, wv, bv, ws, bs,
                  h_ref, q_ref, k_ref, vp_ref, xs1_ref):
    aggs = agg_ref[0] + agg_ref[1]
    num = aggs[:, :DD]
    den = aggs[:, DD:DD + 1]
    h = jnp.maximum(num / (den + 1e-16) + xs_ref[...], 0.0)
    h_ref[...] = h
    q_ref[...] = jnp.dot(h, wq[...], preferred_element_type=jnp.float32) + bq[...]
    k_ref[...] = jnp.dot(h, wk[...], preferred_element_type=jnp.float32) + bk[...]
    v = jnp.dot(h, wv[...], preferred_element_type=jnp.float32) + bv[...]
    pad = (lax.broadcasted_iota(jnp.int32, (h.shape[0], VP - DD), 1) == 0
           ).astype(jnp.float32)
    vp_ref[...] = jnp.concatenate([v, pad], axis=1)
    xs1_ref[...] = jnp.dot(h, ws[...], preferred_element_type=jnp.float32) + bs[...]


def _combine_project(agg, xs, wq, bq, wk, bk, wv, bv, ws, bs):
    nb = 10
    blk = NN // nb
    wspec = pl.BlockSpec((DD, DD), lambda i: (0, 0))
    bspec = pl.BlockSpec((1, DD), lambda i: (0, 0))
    rspec = pl.BlockSpec((blk, DD), lambda i: (i, 0))
    aspec = pl.BlockSpec((NC, blk, VP), lambda i: (0, i, 0))
    return pl.pallas_call(
        _combine_body,
        grid=(nb,),
        in_specs=[aspec, rspec, wspec, bspec, wspec, bspec, wspec, bspec,
                  wspec, bspec],
        out_specs=[rspec, rspec, rspec,
                   pl.BlockSpec((blk, VP), lambda i: (i, 0)), rspec],
        out_shape=[
            jax.ShapeDtypeStruct((NN, DD), jnp.float32),
            jax.ShapeDtypeStruct((NN, DD), jnp.float32),
            jax.ShapeDtypeStruct((NN, DD), jnp.float32),
            jax.ShapeDtypeStruct((NN, VP), jnp.float32),
            jax.ShapeDtypeStruct((NN, DD), jnp.float32),
        ],
    )(agg, xs, wq, bq, wk, bk, wv, bv, ws, bs)


def _final_body(agg_ref, xs_ref, h1_ref, batch_ref, w1, b1, w2, b2,
                out_ref, p1_ref, p2_ref):
    i = pl.program_id(0)
    nb = pl.num_programs(0)
    aggs = agg_ref[0] + agg_ref[1]
    num = aggs[:, :DD]
    den = aggs[:, DD:DD + 1]
    h2 = jnp.maximum(num / (den + 1e-16) + xs_ref[...], 0.0)
    bb = batch_ref[0, 0]
    onehot = (bb[None, :] ==
              lax.broadcasted_iota(jnp.int32, (GG, bb.shape[0]), 0)
              ).astype(jnp.float32)
    p1c = jnp.dot(onehot, h1_ref[...], preferred_element_type=jnp.float32)
    p2c = jnp.dot(onehot, h2, preferred_element_type=jnp.float32)

    @pl.when(i == 0)
    def _():
        p1_ref[...] = p1c
        p2_ref[...] = p2c

    @pl.when(i > 0)
    def _():
        p1_ref[...] = p1_ref[...] + p1c
        p2_ref[...] = p2_ref[...] + p2c

    @pl.when(i == nb - 1)
    def _():
        hh = jnp.maximum(
            jnp.dot(p2_ref[...], w1[...], preferred_element_type=jnp.float32)
            + b1[...], 0.0)
        out_ref[...] = jnp.dot(hh, w2[...],
                               preferred_element_type=jnp.float32) + b2[...]


def _final_stage(agg, xs, h1, batch3, w1, b1, w2, b2):
    nb = 10
    blk = NN // nb
    rspec = pl.BlockSpec((blk, DD), lambda i: (i, 0))
    aspec = pl.BlockSpec((NC, blk, VP), lambda i: (0, i, 0))
    return pl.pallas_call(
        _final_body,
        grid=(nb,),
        in_specs=[aspec, rspec, rspec,
                  pl.BlockSpec((1, 1, blk), lambda i: (i, 0, 0)),
                  pl.BlockSpec((DD, DD), lambda i: (0, 0)),
                  pl.BlockSpec((1, DD), lambda i: (0, 0)),
                  pl.BlockSpec((DD, OUTD), lambda i: (0, 0)),
                  pl.BlockSpec((1, OUTD), lambda i: (0, 0))],
        out_specs=[pl.BlockSpec((GG, OUTD), lambda i: (0, 0)),
                   pl.BlockSpec((GG, DD), lambda i: (0, 0)),
                   pl.BlockSpec((GG, DD), lambda i: (0, 0))],
        out_shape=[
            jax.ShapeDtypeStruct((GG, OUTD), jnp.float32),
            jax.ShapeDtypeStruct((GG, DD), jnp.float32),
            jax.ShapeDtypeStruct((GG, DD), jnp.float32),
        ],
    )(agg, xs, h1, batch3, w1, b1, w2, b2)


def kernel(x, edge_index, batch, Wq0, bq0, Wk0, bk0, Wv0, bv0, Ws0, bs0,
           Wq1, bq1, Wk1, bk1, Wv1, bv1, Ws1, bs1, W1, b1, W2, b2):
    src = edge_index[0].astype(jnp.int32)
    dst = edge_index[1].astype(jnp.int32)
    zeros_hbm = jnp.zeros((NN, VP), jnp.float32)
    batch3 = batch.astype(jnp.int32).reshape(10, 1, NN // 10)
    r = lambda b: b.reshape(1, -1)

    q0, k0, vp0, xs0 = _projections(x, Wq0, r(bq0), Wk0, r(bk0),
                                    Wv0, r(bv0), Ws0, r(bs0))
    agg0 = _get_sc_edge_pass()(q0, k0, vp0, src, dst, zeros_hbm)
    h1, q1, k1, vp1, xs1 = _combine_project(agg0, xs0, Wq1, r(bq1), Wk1,
                                            r(bk1), Wv1, r(bv1), Ws1, r(bs1))
    agg1 = _get_sc_edge_pass()(q1, k1, vp1, src, dst, zeros_hbm)
    out, p1, p2 = _final_stage(agg1, xs1, h1, batch3, W1, r(b1), W2, r(b2))
    return (out, p1, p2)
